# Initial kernel scaffold; baseline (speedup 1.0000x reference)
import jax
import jax.numpy as jnp
from jax.experimental import pallas as pl


def kernel(x_nchw, conv1_w, conv1_b, bn1_gamma, bn1_beta, bn1_mean, bn1_var, se_w1, se_w2, conv2a_w, conv2a_b, bn2a_gamma, bn2a_beta, bn2a_mean, bn2a_var, conv2b_w, conv2b_b, bn2b_gamma, bn2b_beta, bn2b_mean, bn2b_var):
    raise NotImplementedError("write your pallas kernel here")


# speedup vs baseline: 1.6477x; 1.6477x over previous
import functools

import numpy as np
import jax
import jax.numpy as jnp
from jax.experimental import pallas as pl
from jax.experimental.pallas import tpu as pltpu

_EPS = 1e-5


def _rot_lanes_bf16(x, shift_left):
    n = x.shape[-1]
    if shift_left % n == 0:
        return x
    xi = pltpu.bitcast(x, jnp.int32)
    ri = pltpu.roll(xi, shift=(-shift_left) % n, axis=1)
    return pltpu.bitcast(ri, jnp.bfloat16)


def _msff_body(x_ref, mask_ref,
               w1_ref, s1_ref, b1_ref,
               se1_ref, se2_ref,
               w2a_ref, s2a_ref, b2a_ref,
               w2b_ref, s2b_ref, b2b_ref,
               out_ref,
               p0_ref, p1_ref,
               *, H, W, bblk, C, Ch):
    HW = H * W
    mask = mask_ref[...]
    conv_idx = [0]

    def conv3x3(act_bf, cin, w, scale, bias):
        patches_ref = (p0_ref, p1_ref)[conv_idx[0] % 2]
        conv_idx[0] += 1
        for k in range(9):
            dy, dx = divmod(k, 3)
            s = (dy - 1) * W + (dx - 1)
            r0 = k * cin
            if s == 0:
                patches_ref[r0:r0 + cin, :] = act_bf
            else:
                patches_ref[r0:r0 + cin, :] = (
                    _rot_lanes_bf16(act_bf, s) * mask[k:k + 1, :])
        y = jnp.dot(w, patches_ref[0:9 * cin, :],
                    preferred_element_type=jnp.float32)
        return jnp.maximum(y * scale + bias, 0.0)

    w1 = w1_ref[...]
    w2a = w2a_ref[...]
    w2b = w2b_ref[...]
    s1, b1 = s1_ref[...], b1_ref[...]
    s2a, b2a = s2a_ref[...], b2a_ref[...]
    s2b, b2b = s2b_ref[...], b2b_ref[...]
    se1 = se1_ref[...]
    se2 = se2_ref[...]

    for b in range(bblk):
        x = x_ref[b]
        x_bf = x.astype(jnp.bfloat16)

        y1 = conv3x3(x_bf, C, w1, s1, b1)

        pooled = jnp.mean(x, axis=1, keepdims=True)
        hid = jnp.maximum(
            jnp.sum(se1 * pooled, axis=0, keepdims=True), 0.0)
        att = jax.nn.sigmoid(
            jnp.sum(se2 * hid, axis=1, keepdims=True))
        m = y1 * (x * att)

        y2 = conv3x3(m.astype(jnp.bfloat16), C, w2a, s2a, b2a)
        y3 = conv3x3(y2.astype(jnp.bfloat16), Ch, w2b, s2b, b2b)
        out_ref[b] = y3


def _flat_w(w_oihw):
    cout, cin = w_oihw.shape[:2]
    return jnp.transpose(w_oihw.astype(jnp.float32), (0, 2, 3, 1)).reshape(
        cout, 9 * cin).astype(jnp.bfloat16)


def _fold_bn(conv_b, gamma, beta, mean, var):
    scale = gamma / jnp.sqrt(var + _EPS)
    bias = beta + (conv_b - mean) * scale
    return scale[:, None], bias[:, None]


@functools.lru_cache(maxsize=None)
def _np_border_mask(H, W):
    hh, ww = np.meshgrid(np.arange(H), np.arange(W), indexing="ij")
    rows = []
    for dy in range(3):
        for dx in range(3):
            v = ((hh + dy - 1 >= 0) & (hh + dy - 1 < H) &
                 (ww + dx - 1 >= 0) & (ww + dx - 1 < W))
            rows.append(v.reshape(-1))
    m = np.zeros((16, H * W), np.float32)
    m[:9] = np.stack(rows)
    return m


def kernel(x_nchw, conv1_w, conv1_b, bn1_gamma, bn1_beta, bn1_mean, bn1_var,
           se_w1, se_w2,
           conv2a_w, conv2a_b, bn2a_gamma, bn2a_beta, bn2a_mean, bn2a_var,
           conv2b_w, conv2b_b, bn2b_gamma, bn2b_beta, bn2b_mean, bn2b_var):
    B, C, H, W = x_nchw.shape
    Ch, Cr, HW = C // 2, C // 4, H * W
    Bblk = 4 if B % 4 == 0 else 1
    nsteps = B // Bblk

    x = x_nchw.astype(jnp.float32).reshape(B, C, HW)

    w1 = _flat_w(conv1_w)
    s1, b1 = _fold_bn(conv1_b, bn1_gamma, bn1_beta, bn1_mean, bn1_var)
    w2a = _flat_w(conv2a_w)
    s2a, b2a = _fold_bn(conv2a_b, bn2a_gamma, bn2a_beta, bn2a_mean, bn2a_var)
    w2b = _flat_w(conv2b_w)
    s2b, b2b = _fold_bn(conv2b_b, bn2b_gamma, bn2b_beta, bn2b_mean, bn2b_var)
    se1 = se_w1.astype(jnp.float32)
    se2 = jnp.transpose(se_w2).astype(jnp.float32)
    mask = jnp.asarray(_np_border_mask(H, W), jnp.bfloat16)

    def fixed(shape):
        return pl.BlockSpec(shape, lambda b: (0,) * len(shape))

    body = functools.partial(_msff_body, H=H, W=W, bblk=Bblk, C=C, Ch=Ch)
    out = pl.pallas_call(
        body,
        out_shape=jax.ShapeDtypeStruct((B, Ch, HW), jnp.float32),
        grid=(nsteps,),
        in_specs=[
            pl.BlockSpec((Bblk, C, HW), lambda b: (b, 0, 0)),
            fixed((16, HW)),
            fixed((C, 9 * C)), fixed((C, 1)), fixed((C, 1)),
            fixed((C, Cr)), fixed((C, Cr)),
            fixed((Ch, 9 * C)), fixed((Ch, 1)), fixed((Ch, 1)),
            fixed((Ch, 9 * Ch)), fixed((Ch, 1)), fixed((Ch, 1)),
        ],
        out_specs=pl.BlockSpec((Bblk, Ch, HW), lambda b: (b, 0, 0)),
        scratch_shapes=[pltpu.VMEM((9 * C, HW), jnp.bfloat16),
                        pltpu.VMEM((9 * C, HW), jnp.bfloat16)],
        compiler_params=pltpu.CompilerParams(
            dimension_semantics=("parallel",)),
    )(x, mask, w1, s1, b1, se1, se2, w2a, s2a, b2a, w2b, s2b, b2b)

    return out.reshape(B, Ch, H, W)
